# baseline (speedup 1.0000x reference)
import functools

import jax
import jax.numpy as jnp
from jax import lax
from jax.experimental import pallas as pl
from jax.experimental.pallas import tpu as pltpu
from jax.experimental.pallas import tpu_sc as plsc

BATCH = 16384
HIST = 50
DIM = 16
NUM_CORES = 2
NUM_SUBCORES = 16
NW = NUM_CORES * NUM_SUBCORES
B = BATCH * HIST
BPW = B // NW
CHUNK = 3200
NCHUNK = BPW // CHUNK

_mesh = plsc.VectorSubcoreMesh(core_axis_name="c", subcore_axis_name="s")


@functools.partial(
    pl.kernel,
    mesh=_mesh,
    compiler_params=pltpu.CompilerParams(use_tc_tiling_on_sc=False),
    out_type=jax.ShapeDtypeStruct((B, DIM), jnp.float32),
    scratch_types=[
        pltpu.VMEM((BPW,), jnp.int32),
        pltpu.VMEM((CHUNK, DIM), jnp.float32),
        pltpu.SemaphoreType.DMA,
    ],
)
def _sc_gather(idx_hbm, table_hbm, out_hbm, idx_v, rows_v, gsem):
    wid = lax.axis_index("s") * NUM_CORES + lax.axis_index("c")
    base = wid * BPW
    pltpu.sync_copy(idx_hbm.at[pl.ds(base, BPW)], idx_v)
    for i in range(NCHUNK):
        pltpu.async_copy(
            table_hbm.at[idx_v.at[pl.ds(i * CHUNK, CHUNK)]], rows_v, gsem
        ).wait()
        pltpu.sync_copy(rows_v, out_hbm.at[pl.ds(base + i * CHUNK, CHUNK)])


def kernel(indices, table):
    flat_idx = indices.reshape(B)
    out = _sc_gather(flat_idx, table)
    return out.reshape(BATCH, HIST, DIM)


# speedup vs baseline: 1.0044x; 1.0044x over previous
import functools

import jax
import jax.numpy as jnp
from jax import lax
from jax.experimental import pallas as pl
from jax.experimental.pallas import tpu as pltpu
from jax.experimental.pallas import tpu_sc as plsc

BATCH = 16384
HIST = 50
DIM = 16
NUM_CORES = 2
NUM_SUBCORES = 16
NW = NUM_CORES * NUM_SUBCORES
B = BATCH * HIST
BPW = B // NW
CHUNK = 2560
NCHUNK = BPW // CHUNK

_mesh = plsc.VectorSubcoreMesh(core_axis_name="c", subcore_axis_name="s")


@functools.partial(
    pl.kernel,
    mesh=_mesh,
    compiler_params=pltpu.CompilerParams(use_tc_tiling_on_sc=False),
    out_type=jax.ShapeDtypeStruct((B, DIM), jnp.float32),
    scratch_types=[
        pltpu.VMEM((BPW,), jnp.int32),
        pltpu.VMEM((CHUNK, DIM), jnp.float32),
        pltpu.VMEM((CHUNK, DIM), jnp.float32),
        pltpu.SemaphoreType.DMA,
        pltpu.SemaphoreType.DMA,
        pltpu.SemaphoreType.DMA,
        pltpu.SemaphoreType.DMA,
    ],
)
def _sc_gather(idx_hbm, table_hbm, out_hbm, idx_v,
               rows0, rows1, gsem0, gsem1, wsem0, wsem1):
    wid = lax.axis_index("s") * NUM_CORES + lax.axis_index("c")
    base = wid * BPW
    pltpu.sync_copy(idx_hbm.at[pl.ds(base, BPW)], idx_v)

    rows = (rows0, rows1)
    gsem = (gsem0, gsem1)
    wsem = (wsem0, wsem1)

    def start_gather(i):
        b = i % 2
        return pltpu.async_copy(
            table_hbm.at[idx_v.at[pl.ds(i * CHUNK, CHUNK)]], rows[b], gsem[b]
        )

    def start_write(i):
        b = i % 2
        return pltpu.async_copy(
            rows[b], out_hbm.at[pl.ds(base + i * CHUNK, CHUNK)], wsem[b]
        )

    gh = start_gather(0)
    wh = [None, None]
    for i in range(NCHUNK):
        nh = None
        if i + 1 < NCHUNK:
            nb = (i + 1) % 2
            if wh[nb] is not None:
                wh[nb].wait()
            nh = start_gather(i + 1)
        gh.wait()
        wh[i % 2] = start_write(i)
        gh = nh
    wh[(NCHUNK - 1) % 2].wait()
    if wh[NCHUNK % 2] is not None:
        wh[NCHUNK % 2].wait()


def kernel(indices, table):
    flat_idx = indices.reshape(B)
    out = _sc_gather(flat_idx, table)
    return out.reshape(BATCH, HIST, DIM)


# speedup vs baseline: 1.0050x; 1.0006x over previous
import functools

import jax
import jax.numpy as jnp
from jax import lax
from jax.experimental import pallas as pl
from jax.experimental.pallas import tpu as pltpu
from jax.experimental.pallas import tpu_sc as plsc

BATCH = 16384
HIST = 50
DIM = 16
VOCAB = 1000000
NUM_CORES = 2
NUM_SUBCORES = 16
NW = NUM_CORES * NUM_SUBCORES
BATCH_PW = BATCH // NW
CB = 64
NCHUNK = BATCH_PW // CB

_mesh = plsc.VectorSubcoreMesh(core_axis_name="c", subcore_axis_name="s")


@functools.partial(
    pl.kernel,
    mesh=_mesh,
    compiler_params=pltpu.CompilerParams(use_tc_tiling_on_sc=False),
    out_type=jax.ShapeDtypeStruct((BATCH * HIST, DIM), jnp.float32),
    scratch_types=[
        pltpu.VMEM((BATCH_PW * HIST,), jnp.int32),
        pltpu.VMEM((CB * HIST, DIM), jnp.float32),
        pltpu.VMEM((CB * HIST, DIM), jnp.float32),
        pltpu.SemaphoreType.DMA,
        pltpu.SemaphoreType.DMA,
        pltpu.SemaphoreType.DMA,
        pltpu.SemaphoreType.DMA,
    ],
)
def _sc_gather(idx_hbm, table_hbm, out_hbm, idx_v,
               rows0, rows1, gsem0, gsem1, wsem0, wsem1):
    wid = lax.axis_index("s") * NUM_CORES + lax.axis_index("c")
    bbase = wid * BATCH_PW
    pltpu.sync_copy(idx_hbm.at[pl.ds(wid * BATCH_PW * HIST, BATCH_PW * HIST)],
                    idx_v)

    rows = (rows0, rows1)
    gsem = (gsem0, gsem1)
    wsem = (wsem0, wsem1)

    def start_gather(i):
        b = i % 2
        return pltpu.async_copy(
            table_hbm.at[idx_v.at[pl.ds(i * CB * HIST, CB * HIST)]],
            rows[b],
            gsem[b],
        )

    def start_write(i):
        b = i % 2
        return pltpu.async_copy(
            rows[b],
            out_hbm.at[pl.ds((bbase + i * CB) * HIST, CB * HIST)],
            wsem[b],
        )

    gh = start_gather(0)
    wh = [None, None]
    for i in range(NCHUNK):
        nh = None
        if i + 1 < NCHUNK:
            nb = (i + 1) % 2
            if wh[nb] is not None:
                wh[nb].wait()
            nh = start_gather(i + 1)
        gh.wait()
        wh[i % 2] = start_write(i)
        gh = nh
    wh[(NCHUNK - 1) % 2].wait()
    if wh[NCHUNK % 2] is not None:
        wh[NCHUNK % 2].wait()


def kernel(indices, table):
    flat_idx = indices.reshape(BATCH * HIST)
    table_lin = jax.lax.optimization_barrier(table.reshape(VOCAB * DIM))
    out = _sc_gather(flat_idx, table_lin.reshape(VOCAB, DIM))
    return jax.lax.optimization_barrier(out).reshape(BATCH, HIST, DIM)


# speedup vs baseline: 1.0051x; 1.0001x over previous
import functools

import jax
import jax.numpy as jnp
from jax import lax
from jax.experimental import pallas as pl
from jax.experimental.pallas import tpu as pltpu
from jax.experimental.pallas import tpu_sc as plsc

BATCH = 16384
HIST = 50
DIM = 16
VOCAB = 1000000
NUM_CORES = 2
NUM_SUBCORES = 16
NW = NUM_CORES * NUM_SUBCORES
B = BATCH * HIST
BPW = B // NW
CHUNK = 3200
NCHUNK = BPW // CHUNK

_mesh = plsc.VectorSubcoreMesh(core_axis_name="c", subcore_axis_name="s")


@functools.partial(
    pl.kernel,
    mesh=_mesh,
    compiler_params=pltpu.CompilerParams(
        use_tc_tiling_on_sc=False, needs_layout_passes=False
    ),
    out_type=jax.ShapeDtypeStruct((B, DIM), jnp.float32),
    scratch_types=[
        pltpu.VMEM((BPW,), jnp.int32),
        pltpu.VMEM((CHUNK, DIM), jnp.float32),
        pltpu.VMEM((CHUNK, DIM), jnp.float32),
        pltpu.SemaphoreType.DMA,
        pltpu.SemaphoreType.DMA,
        pltpu.SemaphoreType.DMA,
        pltpu.SemaphoreType.DMA,
    ],
)
def _sc_gather(idx_hbm, table_hbm, out_hbm, idx_v,
               rows0, rows1, gsem0, gsem1, wsem0, wsem1):
    wid = lax.axis_index("s") * NUM_CORES + lax.axis_index("c")
    base = wid * BPW
    pltpu.sync_copy(idx_hbm.at[pl.ds(base, BPW)], idx_v)

    rows = (rows0, rows1)
    gsem = (gsem0, gsem1)
    wsem = (wsem0, wsem1)

    def start_gather(i):
        b = i % 2
        return pltpu.async_copy(
            table_hbm.at[idx_v.at[pl.ds(i * CHUNK, CHUNK)]], rows[b], gsem[b]
        )

    def start_write(i):
        b = i % 2
        return pltpu.async_copy(
            rows[b], out_hbm.at[pl.ds(base + i * CHUNK, CHUNK)], wsem[b]
        )

    gh = start_gather(0)
    wh = [None, None]
    for i in range(NCHUNK):
        nh = None
        if i + 1 < NCHUNK:
            nb = (i + 1) % 2
            if wh[nb] is not None:
                wh[nb].wait()
            nh = start_gather(i + 1)
        gh.wait()
        wh[i % 2] = start_write(i)
        gh = nh
    wh[(NCHUNK - 1) % 2].wait()
    if wh[NCHUNK % 2] is not None:
        wh[NCHUNK % 2].wait()


def kernel(indices, table):
    flat_idx = indices.reshape(B)
    out = _sc_gather(flat_idx, table)
    return out.reshape(BATCH, HIST, DIM)


# speedup vs baseline: 1.4688x; 1.4613x over previous
import functools

import jax
import jax.numpy as jnp
from jax import lax
from jax.experimental import pallas as pl
from jax.experimental.pallas import tpu as pltpu
from jax.experimental.pallas import tpu_sc as plsc

BATCH = 16384
HIST = 50
DIM = 16
VOCAB = 1000000
NUM_CORES = 2
NUM_SUBCORES = 16
NW = NUM_CORES * NUM_SUBCORES
BATCH_PW = BATCH // NW
CB = 32
CHUNK = CB * HIST
NCHUNK = BATCH_PW // CB

_mesh = plsc.VectorSubcoreMesh(core_axis_name="c", subcore_axis_name="s")


@functools.partial(
    pl.kernel,
    mesh=_mesh,
    compiler_params=pltpu.CompilerParams(
        use_tc_tiling_on_sc=False, needs_layout_passes=False
    ),
    out_type=jax.ShapeDtypeStruct((BATCH, DIM, HIST), jnp.float32),
    scratch_types=[
        pltpu.VMEM((BATCH_PW * HIST // 2,), jnp.int32),
        pltpu.VMEM((CHUNK, DIM), jnp.float32),
        pltpu.VMEM((CHUNK, DIM), jnp.float32),
        pltpu.VMEM((CB, DIM, HIST), jnp.float32),
        pltpu.VMEM((CB, DIM, HIST), jnp.float32),
        pltpu.SemaphoreType.DMA,
        pltpu.SemaphoreType.DMA,
        pltpu.SemaphoreType.DMA,
        pltpu.SemaphoreType.DMA,
    ],
)
def _sc_gather(idx_hbm, table_hbm, out_hbm, idx_v,
               rows0, rows1, tb0, tb1, gsem0, gsem1, wsem0, wsem1):
    wid = lax.axis_index("s") * NUM_CORES + lax.axis_index("c")
    bbase = wid * BATCH_PW
    half = BATCH_PW * HIST // 2
    pltpu.sync_copy(idx_hbm.at[pl.ds(wid * BATCH_PW * HIST, half)], idx_v)

    rows = (rows0, rows1)
    tbuf = (tb0, tb1)
    gsem = (gsem0, gsem1)
    wsem = (wsem0, wsem1)
    lanes = lax.iota(jnp.int32, 16)

    def start_gather(i):
        b = i % 2
        return pltpu.async_copy(
            table_hbm.at[idx_v.at[pl.ds((i * CHUNK) % half, CHUNK)]],
            rows[b],
            gsem[b],
        )

    def start_write(i):
        b = i % 2
        return pltpu.async_copy(
            tbuf[b], out_hbm.at[pl.ds(bbase + i * CB, CB)], wsem[b]
        )

    def transpose_chunk(b):
        rb = rows[b]
        tb = tbuf[b]

        def body(jj, carry):
            jv = jnp.full((16,), jj, jnp.int32)
            for l in range(HIST):
                vals = rb[jj * HIST + l, :]
                plsc.store_scatter(
                    tb, [jv, lanes, jnp.full((16,), l, jnp.int32)], vals
                )
            return carry

        lax.fori_loop(0, CB, body, 0)

    gh = start_gather(0)
    wh = [None, None]
    for i in range(NCHUNK):
        if i + 1 == NCHUNK // 2:
            gh.wait()
            gh = None
            pltpu.sync_copy(
                idx_hbm.at[pl.ds(wid * BATCH_PW * HIST + half, half)], idx_v
            )
        nh = start_gather(i + 1) if i + 1 < NCHUNK else None
        if gh is not None:
            gh.wait()
        b = i % 2
        if wh[b] is not None:
            wh[b].wait()
        transpose_chunk(b)
        wh[b] = start_write(i)
        gh = nh
    wh[0].wait()
    wh[1].wait()


def kernel(indices, table):
    flat_idx = indices.reshape(BATCH * HIST)
    out = _sc_gather(flat_idx, table)
    return out.transpose(0, 2, 1)
